# baseline (speedup 1.0000x reference)
import functools

import jax
import jax.numpy as jnp
from jax import lax
from jax.experimental import pallas as pl
from jax.experimental.pallas import tpu as pltpu
from jax.experimental.pallas import tpu_sc as plsc

N = 10000
D = 128
NT = 16
CHUNK = 64
BLOCK = 16
SLOTS = 5
STRIPE = 632
NPAD = NT * STRIPE


def _sc_agg_body(nj, gather,
                 table0, src0, dst0, table1, src1, dst1, z128, ones_h,
                 out0, out1,
                 sidx, didx, rows, acc, gsem, ssem):
    cid = lax.axis_index("c")
    sid = lax.axis_index("s")
    row0 = sid * STRIPE
    pltpu.sync_copy(z128, acc.at[pl.ds(row0, STRIPE)])
    if not gather:
        pltpu.sync_copy(ones_h, rows.at[0])
    plsc.subcore_barrier()

    def run(table_h, src_h, dst_h):
        def g_copy(jj, b):
            return pltpu.make_async_copy(table_h.at[sidx.at[jj]], rows.at[b],
                                         gsem.at[b])

        def blk(h, carry):
            base = sid * nj + h * BLOCK
            if gather:
                pltpu.sync_copy(src_h.at[pl.ds(base, BLOCK)], sidx)
            pltpu.sync_copy(dst_h.at[pl.ds(base, BLOCK)], didx)
            if gather:
                for b in range(SLOTS):
                    g_copy(b, b).start()
                for j in range(BLOCK):
                    b = j % SLOTS
                    g_copy(j, b).wait()
                    pltpu.sync_copy(rows.at[b], acc.at[didx.at[j]], add=True)
                    if j + SLOTS < BLOCK:
                        g_copy(j + SLOTS, b).start()
            else:
                for j in range(BLOCK):
                    pltpu.sync_copy(rows.at[0], acc.at[didx.at[j]], add=True)
            return carry

        lax.fori_loop(0, nj // BLOCK, blk, 0)

    @pl.when(cid == 0)
    def _():
        run(table0, src0, dst0)

    @pl.when(cid == 1)
    def _():
        run(table1, src1, dst1)

    plsc.subcore_barrier()

    @pl.when(cid == 0)
    def _():
        pltpu.sync_copy(acc.at[pl.ds(row0, STRIPE)], out0.at[pl.ds(row0, STRIPE)])

    @pl.when(cid == 1)
    def _():
        pltpu.sync_copy(acc.at[pl.ds(row0, STRIPE)], out1.at[pl.ds(row0, STRIPE)])


@functools.lru_cache(maxsize=None)
def _make_sc_agg(nj, gather):
    mesh = plsc.VectorSubcoreMesh(core_axis_name="c", subcore_axis_name="s",
                                  num_cores=2, num_subcores=NT)
    out_type = (
        jax.ShapeDtypeStruct((NPAD, D), jnp.float32),
        jax.ShapeDtypeStruct((NPAD, D), jnp.float32),
    )
    scratch = [
        pltpu.VMEM((BLOCK, CHUNK), jnp.int32),
        pltpu.VMEM((BLOCK, CHUNK), jnp.int32),
        pltpu.VMEM((SLOTS, CHUNK, D), jnp.float32),
        pltpu.VMEM_SHARED((NPAD, D), jnp.float32),
        pltpu.SemaphoreType.DMA((SLOTS,)),
        pltpu.SemaphoreType.DMA((2,)),
    ]
    return pl.kernel(
        functools.partial(_sc_agg_body, nj, gather),
        out_type=out_type, mesh=mesh, scratch_types=scratch,
        name=f"sc_seg_sum_nj{nj}_{int(gather)}",
    )


def _prep_edges(src, dst):
    e = src.shape[0]
    unit = NT * CHUNK * BLOCK
    ep = -(-e // unit) * unit
    if ep != e:
        pad = ep - e
        src = jnp.concatenate([src, jnp.zeros((pad,), jnp.int32)])
        dst = jnp.concatenate([dst, jnp.full((pad,), N, jnp.int32)])
    return src.reshape(ep // CHUNK, CHUNK), dst.reshape(ep // CHUNK, CHUNK)


def _sc_agg(table0, src0, dst0, table1, src1, dst1, gather):
    s0, d0 = _prep_edges(src0, dst0)
    s1, d1 = _prep_edges(src1, dst1)
    assert s0.shape == s1.shape
    nj = s0.shape[0] // NT
    z128 = jnp.zeros((STRIPE, D), jnp.float32)
    ones = jnp.ones((CHUNK, D), jnp.float32)
    fn = _make_sc_agg(nj, gather)
    return fn(table0, s0, d0, table1, s1, d1, z128, ones)


_BM = 1000


def _dense_body(sum_ref, cnt_ref, x_ref, wl_ref, b_ref, wr_ref, o_ref):
    inv = 1.0 / jnp.maximum(cnt_ref[:, 0:1], 1.0)
    mean = sum_ref[...] * inv
    h = jnp.dot(mean, wl_ref[...], preferred_element_type=jnp.float32)
    h = h + jnp.dot(x_ref[...], wr_ref[...], preferred_element_type=jnp.float32)
    h = h + b_ref[...]
    o_ref[...] = jnp.where(h >= 0, h, 0.01 * h)


def _dense_update(summed, cnt, x, wl, b, wr):
    grid = N // _BM
    return pl.pallas_call(
        _dense_body,
        grid=(grid,),
        in_specs=[
            pl.BlockSpec((_BM, D), lambda i: (i, 0)),
            pl.BlockSpec((_BM, D), lambda i: (i, 0)),
            pl.BlockSpec((_BM, D), lambda i: (i, 0)),
            pl.BlockSpec((D, D), lambda i: (0, 0)),
            pl.BlockSpec((1, D), lambda i: (0, 0)),
            pl.BlockSpec((D, D), lambda i: (0, 0)),
        ],
        out_specs=pl.BlockSpec((_BM, D), lambda i: (i, 0)),
        out_shape=jax.ShapeDtypeStruct((N, D), jnp.float32),
        name="sage_dense",
    )(summed, cnt, x, wl, b.reshape(1, D), wr)


def _final_body(s0_ref, s1_ref, cnt_ref, x_ref, wl_ref, b_ref, wr_ref,
                wo_ref, bo_ref, o_ref):
    inv = 1.0 / jnp.maximum(cnt_ref[:, 0:1], 1.0)
    mean = (s0_ref[...] + s1_ref[...]) * inv
    h = jnp.dot(mean, wl_ref[...], preferred_element_type=jnp.float32)
    h = h + jnp.dot(x_ref[...], wr_ref[...], preferred_element_type=jnp.float32)
    h = h + b_ref[...]
    a2 = jnp.where(h >= 0, h, 0.01 * h)
    o_ref[...] = jnp.dot(a2, wo_ref[...], preferred_element_type=jnp.float32) + bo_ref[...]


def _final_update(sum0, sum1, cnt, x, wl, b, wr, wo, bo):
    grid = N // _BM
    nout = wo.shape[1]
    return pl.pallas_call(
        _final_body,
        grid=(grid,),
        in_specs=[
            pl.BlockSpec((_BM, D), lambda i: (i, 0)),
            pl.BlockSpec((_BM, D), lambda i: (i, 0)),
            pl.BlockSpec((_BM, D), lambda i: (i, 0)),
            pl.BlockSpec((_BM, D), lambda i: (i, 0)),
            pl.BlockSpec((D, D), lambda i: (0, 0)),
            pl.BlockSpec((1, D), lambda i: (0, 0)),
            pl.BlockSpec((D, D), lambda i: (0, 0)),
            pl.BlockSpec((D, nout), lambda i: (0, 0)),
            pl.BlockSpec((1, nout), lambda i: (0, 0)),
        ],
        out_specs=pl.BlockSpec((_BM, nout), lambda i: (i, 0)),
        out_shape=jax.ShapeDtypeStruct((N, nout), jnp.float32),
        name="sage_final",
    )(sum0, sum1, cnt, x, wl, b.reshape(1, D), wr, wo, bo.reshape(1, nout))


def kernel(x_author, x_paper, edge_index_writes, edge_index_rev,
           W_l1_writes_l, b_l1_writes_l, W_l1_writes_r,
           W_l1_rev_l, b_l1_rev_l, W_l1_rev_r,
           W_l2_writes_l, b_l2_writes_l, W_l2_writes_r,
           W_l2_rev_l, b_l2_rev_l, W_l2_rev_r,
           W_out, b_out):
    src_w = edge_index_writes[0].astype(jnp.int32)
    dst_w = edge_index_writes[1].astype(jnp.int32)
    src_r = edge_index_rev[0].astype(jnp.int32)
    dst_r = edge_index_rev[1].astype(jnp.int32)

    cnt_p, cnt_a = _sc_agg(x_author, dst_w, dst_w,
                           x_paper, dst_r, dst_r, False)
    sum_p, sum_a = _sc_agg(x_author, src_w, dst_w,
                           x_paper, src_r, dst_r, True)
    p1 = _dense_update(sum_p[:N], cnt_p[:N], x_paper,
                       W_l1_writes_l, b_l1_writes_l, W_l1_writes_r)
    a1 = _dense_update(sum_a[:N], cnt_a[:N], x_author,
                       W_l1_rev_l, b_l1_rev_l, W_l1_rev_r)

    e = src_r.shape[0]
    h = e // 2
    s2a, s2b = _sc_agg(p1, src_r[:h], dst_r[:h],
                       p1, src_r[h:], dst_r[h:], True)
    return _final_update(s2a[:N], s2b[:N], cnt_a[:N], a1,
                         W_l2_rev_l, b_l2_rev_l, W_l2_rev_r, W_out, b_out)


# speedup vs baseline: 1.0524x; 1.0524x over previous
import functools

import jax
import jax.numpy as jnp
from jax import lax
from jax.experimental import pallas as pl
from jax.experimental.pallas import tpu as pltpu
from jax.experimental.pallas import tpu_sc as plsc

N = 10000
D = 128
NT = 16
CHUNK = 128
BLOCK = 16
SLOTS = 2
STRIPE = 632
NPAD = NT * STRIPE


def _sc_agg_body(nj, gather,
                 table0, src0, dst0, table1, src1, dst1, z128, ones_h,
                 out0, out1,
                 sidx, didx, rows, acc, gsem, ssem):
    cid = lax.axis_index("c")
    sid = lax.axis_index("s")
    row0 = sid * STRIPE
    pltpu.sync_copy(z128, acc.at[pl.ds(row0, STRIPE)])
    if not gather:
        pltpu.sync_copy(ones_h, rows.at[0])
    plsc.subcore_barrier()

    def run(table_h, src_h, dst_h):
        nb = nj // BLOCK

        def g_copy(jj, b):
            return pltpu.make_async_copy(table_h.at[sidx.at[jj]], rows.at[b],
                                         gsem.at[b])

        def load_idx(h, par):
            base = sid * nj + h * BLOCK
            if gather:
                pltpu.sync_copy(src_h.at[pl.ds(base, BLOCK)],
                                sidx.at[pl.ds(par * BLOCK, BLOCK)])
            pltpu.sync_copy(dst_h.at[pl.ds(base, BLOCK)],
                            didx.at[pl.ds(par * BLOCK, BLOCK)])

        if gather:
            load_idx(0, 0)
            for b in range(SLOTS):
                g_copy(b, b).start()

            def blk(h, carry):
                par = lax.rem(h, 2)
                off = par * BLOCK
                nxt_off = (1 - par) * BLOCK
                for j in range(BLOCK):
                    b = j % SLOTS
                    g_copy(j, b).wait()
                    pltpu.sync_copy(rows.at[b], acc.at[didx.at[off + j]],
                                    add=True)
                    if j + SLOTS < BLOCK:
                        g_copy(off + j + SLOTS, b).start()
                    else:
                        if j + SLOTS == BLOCK:
                            @pl.when(h + 1 < nb)
                            def _():
                                load_idx(h + 1, 1 - par)

                        @pl.when(h + 1 < nb)
                        def _():
                            g_copy(nxt_off + j + SLOTS - BLOCK, b).start()
                return carry

            lax.fori_loop(0, nb, blk, 0)
        else:
            def blk(h, carry):
                load_idx(h, 0)
                for j in range(BLOCK):
                    pltpu.sync_copy(rows.at[0], acc.at[didx.at[j]], add=True)
                return carry

            lax.fori_loop(0, nb, blk, 0)

    @pl.when(cid == 0)
    def _():
        run(table0, src0, dst0)

    @pl.when(cid == 1)
    def _():
        run(table1, src1, dst1)

    plsc.subcore_barrier()

    @pl.when(cid == 0)
    def _():
        pltpu.sync_copy(acc.at[pl.ds(row0, STRIPE)], out0.at[pl.ds(row0, STRIPE)])

    @pl.when(cid == 1)
    def _():
        pltpu.sync_copy(acc.at[pl.ds(row0, STRIPE)], out1.at[pl.ds(row0, STRIPE)])


@functools.lru_cache(maxsize=None)
def _make_sc_agg(nj, gather):
    mesh = plsc.VectorSubcoreMesh(core_axis_name="c", subcore_axis_name="s",
                                  num_cores=2, num_subcores=NT)
    out_type = (
        jax.ShapeDtypeStruct((NPAD, D), jnp.float32),
        jax.ShapeDtypeStruct((NPAD, D), jnp.float32),
    )
    scratch = [
        pltpu.VMEM((2 * BLOCK, CHUNK), jnp.int32),
        pltpu.VMEM((2 * BLOCK, CHUNK), jnp.int32),
        pltpu.VMEM((SLOTS, CHUNK, D), jnp.float32),
        pltpu.VMEM_SHARED((NPAD, D), jnp.float32),
        pltpu.SemaphoreType.DMA((SLOTS,)),
        pltpu.SemaphoreType.DMA((2,)),
    ]
    return pl.kernel(
        functools.partial(_sc_agg_body, nj, gather),
        out_type=out_type, mesh=mesh, scratch_types=scratch,
        name=f"sc_seg_sum_nj{nj}_{int(gather)}",
    )


def _prep_edges(src, dst):
    e = src.shape[0]
    unit = NT * CHUNK * BLOCK
    ep = -(-e // unit) * unit
    if ep != e:
        pad = ep - e
        src = jnp.concatenate([src, jnp.zeros((pad,), jnp.int32)])
        dst = jnp.concatenate([dst, jnp.full((pad,), N, jnp.int32)])
    return src.reshape(ep // CHUNK, CHUNK), dst.reshape(ep // CHUNK, CHUNK)


def _sc_agg(table0, src0, dst0, table1, src1, dst1, gather):
    s0, d0 = _prep_edges(src0, dst0)
    s1, d1 = _prep_edges(src1, dst1)
    assert s0.shape == s1.shape
    nj = s0.shape[0] // NT
    z128 = jnp.zeros((STRIPE, D), jnp.float32)
    ones = jnp.ones((CHUNK, D), jnp.float32)
    fn = _make_sc_agg(nj, gather)
    return fn(table0, s0, d0, table1, s1, d1, z128, ones)


_BM = 1000


def _dense_body(sum_ref, cnt_ref, x_ref, wl_ref, b_ref, wr_ref, o_ref):
    inv = 1.0 / jnp.maximum(cnt_ref[:, 0:1], 1.0)
    mean = sum_ref[...] * inv
    h = jnp.dot(mean, wl_ref[...], preferred_element_type=jnp.float32)
    h = h + jnp.dot(x_ref[...], wr_ref[...], preferred_element_type=jnp.float32)
    h = h + b_ref[...]
    o_ref[...] = jnp.where(h >= 0, h, 0.01 * h)


def _dense_update(summed, cnt, x, wl, b, wr):
    grid = N // _BM
    return pl.pallas_call(
        _dense_body,
        grid=(grid,),
        in_specs=[
            pl.BlockSpec((_BM, D), lambda i: (i, 0)),
            pl.BlockSpec((_BM, D), lambda i: (i, 0)),
            pl.BlockSpec((_BM, D), lambda i: (i, 0)),
            pl.BlockSpec((D, D), lambda i: (0, 0)),
            pl.BlockSpec((1, D), lambda i: (0, 0)),
            pl.BlockSpec((D, D), lambda i: (0, 0)),
        ],
        out_specs=pl.BlockSpec((_BM, D), lambda i: (i, 0)),
        out_shape=jax.ShapeDtypeStruct((N, D), jnp.float32),
        name="sage_dense",
    )(summed, cnt, x, wl, b.reshape(1, D), wr)


def _final_body(s0_ref, s1_ref, cnt_ref, x_ref, wl_ref, b_ref, wr_ref,
                wo_ref, bo_ref, o_ref):
    inv = 1.0 / jnp.maximum(cnt_ref[:, 0:1], 1.0)
    mean = (s0_ref[...] + s1_ref[...]) * inv
    h = jnp.dot(mean, wl_ref[...], preferred_element_type=jnp.float32)
    h = h + jnp.dot(x_ref[...], wr_ref[...], preferred_element_type=jnp.float32)
    h = h + b_ref[...]
    a2 = jnp.where(h >= 0, h, 0.01 * h)
    o_ref[...] = jnp.dot(a2, wo_ref[...], preferred_element_type=jnp.float32) + bo_ref[...]


def _final_update(sum0, sum1, cnt, x, wl, b, wr, wo, bo):
    grid = N // _BM
    nout = wo.shape[1]
    return pl.pallas_call(
        _final_body,
        grid=(grid,),
        in_specs=[
            pl.BlockSpec((_BM, D), lambda i: (i, 0)),
            pl.BlockSpec((_BM, D), lambda i: (i, 0)),
            pl.BlockSpec((_BM, D), lambda i: (i, 0)),
            pl.BlockSpec((_BM, D), lambda i: (i, 0)),
            pl.BlockSpec((D, D), lambda i: (0, 0)),
            pl.BlockSpec((1, D), lambda i: (0, 0)),
            pl.BlockSpec((D, D), lambda i: (0, 0)),
            pl.BlockSpec((D, nout), lambda i: (0, 0)),
            pl.BlockSpec((1, nout), lambda i: (0, 0)),
        ],
        out_specs=pl.BlockSpec((_BM, nout), lambda i: (i, 0)),
        out_shape=jax.ShapeDtypeStruct((N, nout), jnp.float32),
        name="sage_final",
    )(sum0, sum1, cnt, x, wl, b.reshape(1, D), wr, wo, bo.reshape(1, nout))


def kernel(x_author, x_paper, edge_index_writes, edge_index_rev,
           W_l1_writes_l, b_l1_writes_l, W_l1_writes_r,
           W_l1_rev_l, b_l1_rev_l, W_l1_rev_r,
           W_l2_writes_l, b_l2_writes_l, W_l2_writes_r,
           W_l2_rev_l, b_l2_rev_l, W_l2_rev_r,
           W_out, b_out):
    src_w = edge_index_writes[0].astype(jnp.int32)
    dst_w = edge_index_writes[1].astype(jnp.int32)
    src_r = edge_index_rev[0].astype(jnp.int32)
    dst_r = edge_index_rev[1].astype(jnp.int32)

    cnt_p, cnt_a = _sc_agg(x_author, dst_w, dst_w,
                           x_paper, dst_r, dst_r, False)
    sum_p, sum_a = _sc_agg(x_author, src_w, dst_w,
                           x_paper, src_r, dst_r, True)
    p1 = _dense_update(sum_p[:N], cnt_p[:N], x_paper,
                       W_l1_writes_l, b_l1_writes_l, W_l1_writes_r)
    a1 = _dense_update(sum_a[:N], cnt_a[:N], x_author,
                       W_l1_rev_l, b_l1_rev_l, W_l1_rev_r)

    e = src_r.shape[0]
    h = e // 2
    s2a, s2b = _sc_agg(p1, src_r[:h], dst_r[:h],
                       p1, src_r[h:], dst_r[h:], True)
    return _final_update(s2a[:N], s2b[:N], cnt_a[:N], a1,
                         W_l2_rev_l, b_l2_rev_l, W_l2_rev_r, W_out, b_out)


# speedup vs baseline: 1.0547x; 1.0022x over previous
import functools

import jax
import jax.numpy as jnp
from jax import lax
from jax.experimental import pallas as pl
from jax.experimental.pallas import tpu as pltpu
from jax.experimental.pallas import tpu_sc as plsc

N = 10000
D = 128
NT = 16
CHUNK = 128
BLOCK = 16
SLOTS = 2
STRIPE = 632
NPAD = NT * STRIPE


def _sc_agg_body(nj, counts,
                 table0, src0, dst0, table1, src1, dst1, z128, ones_h,
                 *out_and_scratch):
    if counts:
        out0, out1, cnt0, cnt1 = out_and_scratch[:4]
        sidx, didx, rows, acc, gsem, ssem = out_and_scratch[4:]
    else:
        out0, out1 = out_and_scratch[:2]
        sidx, didx, rows, acc, gsem, ssem = out_and_scratch[2:]
    cid = lax.axis_index("c")
    sid = lax.axis_index("s")
    row0 = sid * STRIPE
    pltpu.sync_copy(z128, acc.at[pl.ds(row0, STRIPE)])
    plsc.subcore_barrier()

    def run(table_h, src_h, dst_h):
        nb = nj // BLOCK

        def g_copy(jj, b):
            return pltpu.make_async_copy(table_h.at[sidx.at[jj]], rows.at[b],
                                         gsem.at[b])

        def load_idx(h, par):
            base = sid * nj + h * BLOCK
            pltpu.sync_copy(src_h.at[pl.ds(base, BLOCK)],
                            sidx.at[pl.ds(par * BLOCK, BLOCK)])
            pltpu.sync_copy(dst_h.at[pl.ds(base, BLOCK)],
                            didx.at[pl.ds(par * BLOCK, BLOCK)])

        load_idx(0, 0)
        for b in range(SLOTS):
            g_copy(b, b).start()

        def blk(h, carry):
            par = lax.rem(h, 2)
            off = par * BLOCK
            nxt_off = (1 - par) * BLOCK
            for j in range(BLOCK):
                b = j % SLOTS
                g_copy(j, b).wait()
                pltpu.sync_copy(rows.at[b], acc.at[didx.at[off + j]],
                                add=True)
                if j + SLOTS < BLOCK:
                    g_copy(off + j + SLOTS, b).start()
                else:
                    if j + SLOTS == BLOCK:
                        @pl.when(h + 1 < nb)
                        def _():
                            load_idx(h + 1, 1 - par)

                    @pl.when(h + 1 < nb)
                    def _():
                        g_copy(nxt_off + j + SLOTS - BLOCK, b).start()
            return carry

        lax.fori_loop(0, nb, blk, 0)

    def count_run(dst_h):
        nb = nj // BLOCK

        def blk(h, carry):
            base = sid * nj + h * BLOCK
            pltpu.sync_copy(dst_h.at[pl.ds(base, BLOCK)],
                            didx.at[pl.ds(0, BLOCK)])
            for j in range(BLOCK):
                pltpu.sync_copy(rows.at[0], acc.at[didx.at[j]], add=True)
            return carry

        lax.fori_loop(0, nb, blk, 0)

    @pl.when(cid == 0)
    def _():
        run(table0, src0, dst0)

    @pl.when(cid == 1)
    def _():
        run(table1, src1, dst1)

    plsc.subcore_barrier()

    @pl.when(cid == 0)
    def _():
        pltpu.sync_copy(acc.at[pl.ds(row0, STRIPE)], out0.at[pl.ds(row0, STRIPE)])

    @pl.when(cid == 1)
    def _():
        pltpu.sync_copy(acc.at[pl.ds(row0, STRIPE)], out1.at[pl.ds(row0, STRIPE)])

    if counts:
        pltpu.sync_copy(z128, acc.at[pl.ds(row0, STRIPE)])
        pltpu.sync_copy(ones_h, rows.at[0])
        plsc.subcore_barrier()

        @pl.when(cid == 0)
        def _():
            count_run(dst0)

        @pl.when(cid == 1)
        def _():
            count_run(dst1)

        plsc.subcore_barrier()

        @pl.when(cid == 0)
        def _():
            pltpu.sync_copy(acc.at[pl.ds(row0, STRIPE)],
                            cnt0.at[pl.ds(row0, STRIPE)])

        @pl.when(cid == 1)
        def _():
            pltpu.sync_copy(acc.at[pl.ds(row0, STRIPE)],
                            cnt1.at[pl.ds(row0, STRIPE)])


@functools.lru_cache(maxsize=None)
def _make_sc_agg(nj, counts):
    mesh = plsc.VectorSubcoreMesh(core_axis_name="c", subcore_axis_name="s",
                                  num_cores=2, num_subcores=NT)
    n_out = 4 if counts else 2
    out_type = tuple(
        jax.ShapeDtypeStruct((NPAD, D), jnp.float32) for _ in range(n_out))
    scratch = [
        pltpu.VMEM((2 * BLOCK, CHUNK), jnp.int32),
        pltpu.VMEM((2 * BLOCK, CHUNK), jnp.int32),
        pltpu.VMEM((SLOTS, CHUNK, D), jnp.float32),
        pltpu.VMEM_SHARED((NPAD, D), jnp.float32),
        pltpu.SemaphoreType.DMA((SLOTS,)),
        pltpu.SemaphoreType.DMA((2,)),
    ]
    return pl.kernel(
        functools.partial(_sc_agg_body, nj, counts),
        out_type=out_type, mesh=mesh, scratch_types=scratch,
        name=f"sc_seg_sum_nj{nj}_{int(counts)}",
    )


def _prep_edges(src, dst):
    e = src.shape[0]
    unit = NT * CHUNK * BLOCK
    ep = -(-e // unit) * unit
    if ep != e:
        pad = ep - e
        src = jnp.concatenate([src, jnp.zeros((pad,), jnp.int32)])
        dst = jnp.concatenate([dst, jnp.full((pad,), N, jnp.int32)])
    return src.reshape(ep // CHUNK, CHUNK), dst.reshape(ep // CHUNK, CHUNK)


def _sc_agg(table0, src0, dst0, table1, src1, dst1, counts):
    s0, d0 = _prep_edges(src0, dst0)
    s1, d1 = _prep_edges(src1, dst1)
    assert s0.shape == s1.shape
    nj = s0.shape[0] // NT
    z128 = jnp.zeros((STRIPE, D), jnp.float32)
    ones = jnp.ones((CHUNK, D), jnp.float32)
    fn = _make_sc_agg(nj, counts)
    return fn(table0, s0, d0, table1, s1, d1, z128, ones)


_BM = 1000


def _dense_body(sum_ref, cnt_ref, x_ref, wl_ref, b_ref, wr_ref, o_ref):
    inv = 1.0 / jnp.maximum(cnt_ref[:, 0:1], 1.0)
    mean = sum_ref[...] * inv
    h = jnp.dot(mean, wl_ref[...], preferred_element_type=jnp.float32)
    h = h + jnp.dot(x_ref[...], wr_ref[...], preferred_element_type=jnp.float32)
    h = h + b_ref[...]
    o_ref[...] = jnp.where(h >= 0, h, 0.01 * h)


def _dense_update(summed, cnt, x, wl, b, wr):
    grid = N // _BM
    return pl.pallas_call(
        _dense_body,
        grid=(grid,),
        in_specs=[
            pl.BlockSpec((_BM, D), lambda i: (i, 0)),
            pl.BlockSpec((_BM, D), lambda i: (i, 0)),
            pl.BlockSpec((_BM, D), lambda i: (i, 0)),
            pl.BlockSpec((D, D), lambda i: (0, 0)),
            pl.BlockSpec((1, D), lambda i: (0, 0)),
            pl.BlockSpec((D, D), lambda i: (0, 0)),
        ],
        out_specs=pl.BlockSpec((_BM, D), lambda i: (i, 0)),
        out_shape=jax.ShapeDtypeStruct((N, D), jnp.float32),
        name="sage_dense",
    )(summed, cnt, x, wl, b.reshape(1, D), wr)


def _final_body(s0_ref, s1_ref, cnt_ref, x_ref, wl_ref, b_ref, wr_ref,
                wo_ref, bo_ref, o_ref):
    inv = 1.0 / jnp.maximum(cnt_ref[:, 0:1], 1.0)
    mean = (s0_ref[...] + s1_ref[...]) * inv
    h = jnp.dot(mean, wl_ref[...], preferred_element_type=jnp.float32)
    h = h + jnp.dot(x_ref[...], wr_ref[...], preferred_element_type=jnp.float32)
    h = h + b_ref[...]
    a2 = jnp.where(h >= 0, h, 0.01 * h)
    o_ref[...] = jnp.dot(a2, wo_ref[...], preferred_element_type=jnp.float32) + bo_ref[...]


def _final_update(sum0, sum1, cnt, x, wl, b, wr, wo, bo):
    grid = N // _BM
    nout = wo.shape[1]
    return pl.pallas_call(
        _final_body,
        grid=(grid,),
        in_specs=[
            pl.BlockSpec((_BM, D), lambda i: (i, 0)),
            pl.BlockSpec((_BM, D), lambda i: (i, 0)),
            pl.BlockSpec((_BM, D), lambda i: (i, 0)),
            pl.BlockSpec((_BM, D), lambda i: (i, 0)),
            pl.BlockSpec((D, D), lambda i: (0, 0)),
            pl.BlockSpec((1, D), lambda i: (0, 0)),
            pl.BlockSpec((D, D), lambda i: (0, 0)),
            pl.BlockSpec((D, nout), lambda i: (0, 0)),
            pl.BlockSpec((1, nout), lambda i: (0, 0)),
        ],
        out_specs=pl.BlockSpec((_BM, nout), lambda i: (i, 0)),
        out_shape=jax.ShapeDtypeStruct((N, nout), jnp.float32),
        name="sage_final",
    )(sum0, sum1, cnt, x, wl, b.reshape(1, D), wr, wo, bo.reshape(1, nout))


def kernel(x_author, x_paper, edge_index_writes, edge_index_rev,
           W_l1_writes_l, b_l1_writes_l, W_l1_writes_r,
           W_l1_rev_l, b_l1_rev_l, W_l1_rev_r,
           W_l2_writes_l, b_l2_writes_l, W_l2_writes_r,
           W_l2_rev_l, b_l2_rev_l, W_l2_rev_r,
           W_out, b_out):
    src_w = edge_index_writes[0].astype(jnp.int32)
    dst_w = edge_index_writes[1].astype(jnp.int32)
    src_r = edge_index_rev[0].astype(jnp.int32)
    dst_r = edge_index_rev[1].astype(jnp.int32)

    sum_p, sum_a, cnt_p, cnt_a = _sc_agg(x_author, src_w, dst_w,
                                         x_paper, src_r, dst_r, True)
    p1 = _dense_update(sum_p[:N], cnt_p[:N], x_paper,
                       W_l1_writes_l, b_l1_writes_l, W_l1_writes_r)
    a1 = _dense_update(sum_a[:N], cnt_a[:N], x_author,
                       W_l1_rev_l, b_l1_rev_l, W_l1_rev_r)

    e = src_r.shape[0]
    h = e // 2
    s2a, s2b = _sc_agg(p1, src_r[:h], dst_r[:h],
                       p1, src_r[h:], dst_r[h:], False)
    return _final_update(s2a[:N], s2b[:N], cnt_a[:N], a1,
                         W_l2_rev_l, b_l2_rev_l, W_l2_rev_r, W_out, b_out)


# speedup vs baseline: 1.0681x; 1.0127x over previous
import functools

import jax
import jax.numpy as jnp
from jax import lax
from jax.experimental import pallas as pl
from jax.experimental.pallas import tpu as pltpu
from jax.experimental.pallas import tpu_sc as plsc

N = 10000
D = 128
NT = 16
CHUNK = 128
BLOCK = 16
SLOTS = 2
STRIPE = 632
NPAD = NT * STRIPE


def _sc_agg_body(nj, counts,
                 table0, src0, dst0, table1, src1, dst1, z128, ones_h,
                 *out_and_scratch):
    if counts:
        out0, out1, cnt0, cnt1 = out_and_scratch[:4]
        sidx, didx, rows, acc, gsem, ssem = out_and_scratch[4:]
    else:
        out0, out1 = out_and_scratch[:2]
        sidx, didx, rows, acc, gsem, ssem = out_and_scratch[2:]
    cid = lax.axis_index("c")
    sid = lax.axis_index("s")
    row0 = sid * STRIPE
    pltpu.sync_copy(z128, acc.at[pl.ds(row0, STRIPE)])
    plsc.subcore_barrier()

    def run(table_h, src_h, dst_h):
        nb = nj // BLOCK

        def g_copy(jj, b):
            return pltpu.make_async_copy(table_h.at[sidx.at[jj]], rows.at[b],
                                         gsem.at[b])

        def load_idx(h, par):
            base = sid * nj + h * BLOCK
            pltpu.sync_copy(src_h.at[pl.ds(base, BLOCK)],
                            sidx.at[pl.ds(par * BLOCK, BLOCK)])
            pltpu.sync_copy(dst_h.at[pl.ds(base, BLOCK)],
                            didx.at[pl.ds(par * BLOCK, BLOCK)])

        load_idx(0, 0)
        for b in range(SLOTS):
            g_copy(b, b).start()

        def blk(h, carry):
            par = lax.rem(h, 2)
            off = par * BLOCK
            nxt_off = (1 - par) * BLOCK
            for j in range(BLOCK):
                b = j % SLOTS
                g_copy(j, b).wait()
                pltpu.sync_copy(rows.at[b], acc.at[didx.at[off + j]],
                                add=True)
                if j + SLOTS < BLOCK:
                    g_copy(off + j + SLOTS, b).start()
                else:
                    if j + SLOTS == BLOCK:
                        @pl.when(h + 1 < nb)
                        def _():
                            load_idx(h + 1, 1 - par)

                    @pl.when(h + 1 < nb)
                    def _():
                        g_copy(nxt_off + j + SLOTS - BLOCK, b).start()
            return carry

        lax.fori_loop(0, nb, blk, 0)

    def count_run(dst_h):
        nb = nj // BLOCK

        def blk(h, carry):
            base = sid * nj + h * BLOCK
            pltpu.sync_copy(dst_h.at[pl.ds(base, BLOCK)],
                            didx.at[pl.ds(0, BLOCK)])
            for j in range(BLOCK):
                pltpu.sync_copy(rows.at[0], acc.at[didx.at[j]], add=True)
            return carry

        lax.fori_loop(0, nb, blk, 0)

    @pl.when(cid == 0)
    def _():
        run(table0, src0, dst0)

    @pl.when(cid == 1)
    def _():
        run(table1, src1, dst1)

    plsc.subcore_barrier()

    @pl.when(cid == 0)
    def _():
        pltpu.sync_copy(acc.at[pl.ds(row0, STRIPE)], out0.at[pl.ds(row0, STRIPE)])

    @pl.when(cid == 1)
    def _():
        pltpu.sync_copy(acc.at[pl.ds(row0, STRIPE)], out1.at[pl.ds(row0, STRIPE)])

    if counts:
        pltpu.sync_copy(z128, acc.at[pl.ds(row0, STRIPE)])
        pltpu.sync_copy(ones_h, rows.at[0])
        plsc.subcore_barrier()

        @pl.when(cid == 0)
        def _():
            count_run(dst0)

        @pl.when(cid == 1)
        def _():
            count_run(dst1)

        plsc.subcore_barrier()

        @pl.when(cid == 0)
        def _():
            pltpu.sync_copy(acc.at[pl.ds(row0, STRIPE)],
                            cnt0.at[pl.ds(row0, STRIPE)])

        @pl.when(cid == 1)
        def _():
            pltpu.sync_copy(acc.at[pl.ds(row0, STRIPE)],
                            cnt1.at[pl.ds(row0, STRIPE)])


@functools.lru_cache(maxsize=None)
def _make_sc_agg(nj, counts):
    mesh = plsc.VectorSubcoreMesh(core_axis_name="c", subcore_axis_name="s",
                                  num_cores=2, num_subcores=NT)
    n_out = 4 if counts else 2
    out_type = tuple(
        jax.ShapeDtypeStruct((NPAD, D), jnp.float32) for _ in range(n_out))
    scratch = [
        pltpu.VMEM((2 * BLOCK, CHUNK), jnp.int32),
        pltpu.VMEM((2 * BLOCK, CHUNK), jnp.int32),
        pltpu.VMEM((SLOTS, CHUNK, D), jnp.float32),
        pltpu.VMEM_SHARED((NPAD, D), jnp.float32),
        pltpu.SemaphoreType.DMA((SLOTS,)),
        pltpu.SemaphoreType.DMA((2,)),
    ]
    return pl.kernel(
        functools.partial(_sc_agg_body, nj, counts),
        out_type=out_type, mesh=mesh, scratch_types=scratch,
        name=f"sc_seg_sum_nj{nj}_{int(counts)}",
    )


def _prep_edges(src, dst):
    e = src.shape[0]
    unit = NT * CHUNK * BLOCK
    ep = -(-e // unit) * unit
    if ep != e:
        pad = ep - e
        src = jnp.concatenate([src, jnp.zeros((pad,), jnp.int32)])
        dst = jnp.concatenate([dst, jnp.full((pad,), N, jnp.int32)])
    return src.reshape(ep // CHUNK, CHUNK), dst.reshape(ep // CHUNK, CHUNK)


def _sc_agg(table0, s0, d0, table1, s1, d1, counts):
    assert s0.shape == s1.shape
    nj = s0.shape[0] // NT
    z128 = jnp.zeros((STRIPE, D), jnp.float32)
    ones = jnp.ones((CHUNK, D), jnp.float32)
    fn = _make_sc_agg(nj, counts)
    return fn(table0, s0, d0, table1, s1, d1, z128, ones)


_BM = 1000


def _dense_body(sum_ref, cnt_ref, x_ref, wl_ref, b_ref, wr_ref, o_ref,
                o2_ref=None):
    inv = 1.0 / jnp.maximum(cnt_ref[:, 0:1], 1.0)
    mean = sum_ref[...] * inv
    h = jnp.dot(mean, wl_ref[...], preferred_element_type=jnp.float32)
    h = h + jnp.dot(x_ref[...], wr_ref[...], preferred_element_type=jnp.float32)
    h = h + b_ref[...]
    act = jnp.where(h >= 0, h, 0.01 * h)
    o_ref[...] = act
    if o2_ref is not None:
        o2_ref[...] = act


def _dense_update(summed, cnt, x, wl, b, wr, dup=False):
    grid = N // _BM
    n_out = 2 if dup else 1
    out_shape = [jax.ShapeDtypeStruct((N, D), jnp.float32)] * n_out
    out_specs = [pl.BlockSpec((_BM, D), lambda i: (i, 0))] * n_out
    if not dup:
        out_shape, out_specs = out_shape[0], out_specs[0]
    return pl.pallas_call(
        _dense_body,
        grid=(grid,),
        in_specs=[
            pl.BlockSpec((_BM, D), lambda i: (i, 0)),
            pl.BlockSpec((_BM, D), lambda i: (i, 0)),
            pl.BlockSpec((_BM, D), lambda i: (i, 0)),
            pl.BlockSpec((D, D), lambda i: (0, 0)),
            pl.BlockSpec((1, D), lambda i: (0, 0)),
            pl.BlockSpec((D, D), lambda i: (0, 0)),
        ],
        out_specs=out_specs,
        out_shape=out_shape,
        name="sage_dense",
    )(summed, cnt, x, wl, b.reshape(1, D), wr)


def _final_body(s0_ref, s1_ref, cnt_ref, x_ref, wl_ref, b_ref, wr_ref,
                wo_ref, bo_ref, o_ref):
    inv = 1.0 / jnp.maximum(cnt_ref[:, 0:1], 1.0)
    mean = (s0_ref[...] + s1_ref[...]) * inv
    h = jnp.dot(mean, wl_ref[...], preferred_element_type=jnp.float32)
    h = h + jnp.dot(x_ref[...], wr_ref[...], preferred_element_type=jnp.float32)
    h = h + b_ref[...]
    a2 = jnp.where(h >= 0, h, 0.01 * h)
    o_ref[...] = jnp.dot(a2, wo_ref[...], preferred_element_type=jnp.float32) + bo_ref[...]


def _final_update(sum0, sum1, cnt, x, wl, b, wr, wo, bo):
    grid = N // _BM
    nout = wo.shape[1]
    return pl.pallas_call(
        _final_body,
        grid=(grid,),
        in_specs=[
            pl.BlockSpec((_BM, D), lambda i: (i, 0)),
            pl.BlockSpec((_BM, D), lambda i: (i, 0)),
            pl.BlockSpec((_BM, D), lambda i: (i, 0)),
            pl.BlockSpec((_BM, D), lambda i: (i, 0)),
            pl.BlockSpec((D, D), lambda i: (0, 0)),
            pl.BlockSpec((1, D), lambda i: (0, 0)),
            pl.BlockSpec((D, D), lambda i: (0, 0)),
            pl.BlockSpec((D, nout), lambda i: (0, 0)),
            pl.BlockSpec((1, nout), lambda i: (0, 0)),
        ],
        out_specs=pl.BlockSpec((_BM, nout), lambda i: (i, 0)),
        out_shape=jax.ShapeDtypeStruct((N, nout), jnp.float32),
        name="sage_final",
    )(sum0, sum1, cnt, x, wl, b.reshape(1, D), wr, wo, bo.reshape(1, nout))


def kernel(x_author, x_paper, edge_index_writes, edge_index_rev,
           W_l1_writes_l, b_l1_writes_l, W_l1_writes_r,
           W_l1_rev_l, b_l1_rev_l, W_l1_rev_r,
           W_l2_writes_l, b_l2_writes_l, W_l2_writes_r,
           W_l2_rev_l, b_l2_rev_l, W_l2_rev_r,
           W_out, b_out):
    src_w = edge_index_writes[0].astype(jnp.int32)
    dst_w = edge_index_writes[1].astype(jnp.int32)
    src_r = edge_index_rev[0].astype(jnp.int32)
    dst_r = edge_index_rev[1].astype(jnp.int32)

    sw, dw = _prep_edges(src_w, dst_w)
    sr, dr = _prep_edges(src_r, dst_r)

    sum_p, sum_a, cnt_p, cnt_a = _sc_agg(x_author, sw, dw,
                                         x_paper, sr, dr, True)
    p1, p1b = _dense_update(sum_p[:N], cnt_p[:N], x_paper,
                            W_l1_writes_l, b_l1_writes_l, W_l1_writes_r,
                            dup=True)
    a1 = _dense_update(sum_a[:N], cnt_a[:N], x_author,
                       W_l1_rev_l, b_l1_rev_l, W_l1_rev_r)

    hc = sr.shape[0] // 2
    s2a, s2b = _sc_agg(p1, sr[:hc], dr[:hc],
                       p1b, sr[hc:], dr[hc:], False)
    return _final_update(s2a[:N], s2b[:N], cnt_a[:N], a1,
                         W_l2_rev_l, b_l2_rev_l, W_l2_rev_r, W_out, b_out)
